# baseline (speedup 1.0000x reference)
import functools

import jax
import jax.numpy as jnp
from jax import lax
from jax.experimental import pallas as pl
from jax.experimental.pallas import tpu as pltpu
from jax.experimental.pallas import tpu_sc as plsc

N = 10000
E = 320000
H = 128
FIN = 8

NC, NS = 2, 16
NW = NC * NS
C = 96
STEPS = 105
EWP = C * STEPS
EP = EWP * NW
PAD = EP - E
NP = N + 16
HC = C // 2
HALF = (STEPS - 1) // 2
RB = 624
TAIL = N - NS * RB
TAIL0 = NS * RB


@functools.cache
def _mesh():
    return plsc.VectorSubcoreMesh(core_axis_name="c", subcore_axis_name="s",
                                  num_cores=NC, num_subcores=NS)


def _row_init(src_hbm, acc_sh, sid):
    base = sid * RB
    pltpu.sync_copy(src_hbm.at[pl.ds(base, RB)], acc_sh.at[pl.ds(base, RB)])

    @pl.when(sid == NS - 1)
    def _():
        pltpu.sync_copy(src_hbm.at[pl.ds(TAIL0, TAIL)],
                        acc_sh.at[pl.ds(TAIL0, TAIL)])


def _row_out(acc_sh, out_hbm, cid, sid):
    base = sid * RB
    obase = cid * N + sid * RB
    pltpu.sync_copy(acc_sh.at[pl.ds(base, RB)], out_hbm.at[pl.ds(obase, RB)])

    @pl.when(sid == NS - 1)
    def _():
        pltpu.sync_copy(acc_sh.at[pl.ds(TAIL0, TAIL)],
                        out_hbm.at[pl.ds(cid * N + TAIL0, TAIL)])


def _sc_agg_body(hp_hbm, src_hbm, dst_hbm, zeros_hbm, out_hbm,
                 src_a, dst_a, src_b, dst_b, rows_a, rows_b, acc_sh,
                 ia, ib, ga, gb, sa, sb):
    cid = lax.axis_index("c")
    sid = lax.axis_index("s")
    wid = sid * NC + cid
    ebase = wid * EWP

    @pl.when(cid == 0)
    def _():
        _row_init(hp_hbm, acc_sh, sid)

    @pl.when(cid == 1)
    def _():
        _row_init(zeros_hbm, acc_sh, sid)

    def idx_load(k, sv, dv, sem):
        base = ebase + k * C
        pltpu.async_copy(src_hbm.at[pl.ds(base, HC)], sv.at[0], sem)
        pltpu.async_copy(src_hbm.at[pl.ds(base + HC, HC)], sv.at[1], sem)
        pltpu.async_copy(dst_hbm.at[pl.ds(base, HC)], dv.at[0], sem)
        pltpu.async_copy(dst_hbm.at[pl.ds(base + HC, HC)], dv.at[1], sem)

    def idx_wait(sv, dv, sem):
        pltpu.make_async_copy(src_hbm.at[pl.ds(ebase, HC)], sv.at[0], sem).wait()
        pltpu.make_async_copy(src_hbm.at[pl.ds(ebase, HC)], sv.at[1], sem).wait()
        pltpu.make_async_copy(dst_hbm.at[pl.ds(ebase, HC)], dv.at[0], sem).wait()
        pltpu.make_async_copy(dst_hbm.at[pl.ds(ebase, HC)], dv.at[1], sem).wait()

    def gather_start(sv, rows, sem):
        pltpu.async_copy(hp_hbm.at[sv.at[0]], rows.at[0], sem)
        pltpu.async_copy(hp_hbm.at[sv.at[1]], rows.at[1], sem)

    def gather_wait(sv, rows, sem):
        pltpu.make_async_copy(hp_hbm.at[sv.at[0]], rows.at[0], sem).wait()
        pltpu.make_async_copy(hp_hbm.at[sv.at[1]], rows.at[1], sem).wait()

    def scat_start(rows, dv, sem):
        pltpu.async_copy(rows.at[0], acc_sh.at[dv.at[0]], sem, add=True)
        pltpu.async_copy(rows.at[1], acc_sh.at[dv.at[1]], sem, add=True)

    def scat_wait(rows, dv, sem):
        pltpu.make_async_copy(rows.at[0], acc_sh.at[dv.at[0]], sem).wait()
        pltpu.make_async_copy(rows.at[1], acc_sh.at[dv.at[1]], sem).wait()

    plsc.subcore_barrier()

    idx_load(0, src_a, dst_a, ia)
    idx_wait(src_a, dst_a, ia)
    gather_start(src_a, rows_a, ga)
    idx_load(1, src_b, dst_b, ib)

    def body2(j, carry):
        k0 = 2 * j
        gather_wait(src_a, rows_a, ga)
        idx_wait(src_b, dst_b, ib)
        gather_start(src_b, rows_b, gb)
        scat_start(rows_a, dst_a, sa)
        gather_wait(src_b, rows_b, gb)
        scat_start(rows_b, dst_b, sb)
        scat_wait(rows_a, dst_a, sa)
        idx_load(k0 + 2, src_a, dst_a, ia)
        idx_wait(src_a, dst_a, ia)
        gather_start(src_a, rows_a, ga)
        scat_wait(rows_b, dst_b, sb)
        kn = jnp.minimum(k0 + 3, STEPS - 1)
        idx_load(kn, src_b, dst_b, ib)
        return carry

    lax.fori_loop(0, HALF, body2, 0)
    gather_wait(src_a, rows_a, ga)
    scat_start(rows_a, dst_a, sa)
    scat_wait(rows_a, dst_a, sa)
    idx_wait(src_b, dst_b, ib)
    plsc.subcore_barrier()
    _row_out(acc_sh, out_hbm, cid, sid)


def _sc_deg_body(dst_hbm, ones_hbm, zeros_hbm, out_hbm,
                 dst_a, dst_b, ones_v, acc_sh, ia, ib, sa, sb):
    cid = lax.axis_index("c")
    sid = lax.axis_index("s")
    wid = sid * NC + cid
    ebase = wid * EWP

    @pl.when(cid == 0)
    def _():
        _row_init(ones_hbm, acc_sh, sid)

    @pl.when(cid == 1)
    def _():
        _row_init(zeros_hbm, acc_sh, sid)

    pltpu.sync_copy(ones_hbm.at[pl.ds(0, HC)], ones_v)

    def idx_load(k, dv, sem):
        base = ebase + k * C
        pltpu.async_copy(dst_hbm.at[pl.ds(base, HC)], dv.at[0], sem)
        pltpu.async_copy(dst_hbm.at[pl.ds(base + HC, HC)], dv.at[1], sem)

    def idx_wait(dv, sem):
        pltpu.make_async_copy(dst_hbm.at[pl.ds(ebase, HC)], dv.at[0], sem).wait()
        pltpu.make_async_copy(dst_hbm.at[pl.ds(ebase, HC)], dv.at[1], sem).wait()

    def scat_start(dv, sem):
        pltpu.async_copy(ones_v, acc_sh.at[dv.at[0]], sem, add=True)
        pltpu.async_copy(ones_v, acc_sh.at[dv.at[1]], sem, add=True)

    def scat_wait(dv, sem):
        pltpu.make_async_copy(ones_v, acc_sh.at[dv.at[0]], sem).wait()
        pltpu.make_async_copy(ones_v, acc_sh.at[dv.at[1]], sem).wait()

    plsc.subcore_barrier()

    idx_load(0, dst_a, ia)
    idx_wait(dst_a, ia)
    scat_start(dst_a, sa)
    idx_load(1, dst_b, ib)

    def body2(j, carry):
        k0 = 2 * j
        idx_wait(dst_b, ib)
        scat_start(dst_b, sb)
        scat_wait(dst_a, sa)
        idx_load(k0 + 2, dst_a, ia)
        idx_wait(dst_a, ia)
        scat_start(dst_a, sa)
        scat_wait(dst_b, sb)
        kn = jnp.minimum(k0 + 3, STEPS - 1)
        idx_load(kn, dst_b, ib)
        return carry

    lax.fori_loop(0, HALF, body2, 0)
    scat_wait(dst_a, sa)
    idx_wait(dst_b, ib)
    plsc.subcore_barrier()
    _row_out(acc_sh, out_hbm, cid, sid)


@functools.cache
def _sc_agg_call():
    return pl.kernel(
        _sc_agg_body,
        out_type=jax.ShapeDtypeStruct((2 * N, H), jnp.float32),
        mesh=_mesh(),
        scratch_types=[
            pltpu.VMEM((2, HC), jnp.int32),
            pltpu.VMEM((2, HC), jnp.int32),
            pltpu.VMEM((2, HC), jnp.int32),
            pltpu.VMEM((2, HC), jnp.int32),
            pltpu.VMEM((2, HC, H), jnp.float32),
            pltpu.VMEM((2, HC, H), jnp.float32),
            pltpu.VMEM_SHARED((NP, H), jnp.float32),
            pltpu.SemaphoreType.DMA,
            pltpu.SemaphoreType.DMA,
            pltpu.SemaphoreType.DMA,
            pltpu.SemaphoreType.DMA,
            pltpu.SemaphoreType.DMA,
            pltpu.SemaphoreType.DMA,
        ],
    )


def _sc_agg(hp, src, dst, zeros_h):
    return _sc_agg_call()(hp, src, dst, zeros_h)


@functools.cache
def _sc_deg_call():
    return pl.kernel(
        _sc_deg_body,
        out_type=jax.ShapeDtypeStruct((2 * N, H), jnp.float32),
        mesh=_mesh(),
        scratch_types=[
            pltpu.VMEM((2, HC), jnp.int32),
            pltpu.VMEM((2, HC), jnp.int32),
            pltpu.VMEM((HC, H), jnp.float32),
            pltpu.VMEM_SHARED((NP, H), jnp.float32),
            pltpu.SemaphoreType.DMA,
            pltpu.SemaphoreType.DMA,
            pltpu.SemaphoreType.DMA,
            pltpu.SemaphoreType.DMA,
        ],
    )


def _sc_deg(dst, ones_h, zeros_h):
    return _sc_deg_call()(dst, ones_h, zeros_h)


def _tc_pre0_body(x_ref, w_ref, degp_ref, hp_ref, dinv_ref):
    deg = degp_ref[0, :, 0] + degp_ref[1, :, 0]
    dinv = lax.rsqrt(deg)[:, None]
    h = jnp.dot(x_ref[...], w_ref[...], preferred_element_type=jnp.float32)
    hp_ref[...] = h * dinv
    dinv_ref[...] = dinv


def _tc_pre0(x, w0, degp):
    return pl.pallas_call(
        _tc_pre0_body,
        out_shape=(jax.ShapeDtypeStruct((N, H), jnp.float32),
                   jax.ShapeDtypeStruct((N, 1), jnp.float32)),
    )(x, w0, degp)


def _ln_relu(pre, g, be):
    mu = jnp.mean(pre, axis=1, keepdims=True)
    d = pre - mu
    var = jnp.mean(d * d, axis=1, keepdims=True)
    y = d * lax.rsqrt(var + 1e-5) * g + be
    return jnp.maximum(y, 0.0)


def _tc_mid_body(accp_ref, dinv_ref, b_ref, g_ref, be_ref, w_ref, hp_ref):
    s = accp_ref[0] + accp_ref[1]
    pre = s * dinv_ref[...] + b_ref[...]
    y = _ln_relu(pre, g_ref[...], be_ref[...])
    hp_ref[...] = jnp.dot(y, w_ref[...],
                          preferred_element_type=jnp.float32) * dinv_ref[...]


def _tc_mid(accp, dinv, b, g, be, w_next):
    return pl.pallas_call(
        _tc_mid_body,
        out_shape=jax.ShapeDtypeStruct((N, H), jnp.float32),
    )(accp, dinv, b, g, be, w_next)


def _tc_fin_body(accp_ref, dinv_ref, b_ref, g_ref, be_ref, out_ref):
    s = accp_ref[0] + accp_ref[1]
    pre = s * dinv_ref[...] + b_ref[...]
    out_ref[...] = _ln_relu(pre, g_ref[...], be_ref[...])


def _tc_fin(accp, dinv, b, g, be):
    return pl.pallas_call(
        _tc_fin_body,
        out_shape=jax.ShapeDtypeStruct((N, H), jnp.float32),
    )(accp, dinv, b, g, be)


def kernel(node_features, W0, b0, W1, b1, W2, b2,
           g0, be0, g1, be1, g2, be2, edge_index):
    x = node_features.reshape(N, FIN)
    pad_src = jnp.arange(PAD, dtype=jnp.int32) % 16
    pad_dst = N + pad_src
    src = jnp.concatenate([edge_index[0], pad_src])
    dst = jnp.concatenate([edge_index[1], pad_dst])
    zeros_h = jnp.zeros((N, H), jnp.float32)
    ones_h = jnp.ones((N, H), jnp.float32)

    degp = _sc_deg(dst, ones_h, zeros_h).reshape(2, N, H)
    hp, dinv = _tc_pre0(x, W0, degp)

    b0r, b1r, b2r = (v.reshape(1, H) for v in (b0, b1, b2))
    g0r, g1r, g2r = (v.reshape(1, H) for v in (g0, g1, g2))
    be0r, be1r, be2r = (v.reshape(1, H) for v in (be0, be1, be2))

    accp = _sc_agg(hp, src, dst, zeros_h).reshape(2, N, H)
    hp = _tc_mid(accp, dinv, b0r, g0r, be0r, W1)
    accp = _sc_agg(hp, src, dst, zeros_h).reshape(2, N, H)
    hp = _tc_mid(accp, dinv, b1r, g1r, be1r, W2)
    accp = _sc_agg(hp, src, dst, zeros_h).reshape(2, N, H)
    out = _tc_fin(accp, dinv, b2r, g2r, be2r)
    return out.reshape(1, N, H)


# speedup vs baseline: 1.1189x; 1.1189x over previous
import functools

import jax
import jax.numpy as jnp
from jax import lax
from jax.experimental import pallas as pl
from jax.experimental.pallas import tpu as pltpu
from jax.experimental.pallas import tpu_sc as plsc

N = 10000
E = 320000
H = 128
FIN = 8

NC, NS = 2, 16
NW = NC * NS
C = 96
STEPS = 105
EWP = C * STEPS
EP = EWP * NW
PAD = EP - E
NP = N + 16
HC = C // 2
HALF = (STEPS - 1) // 2
RB = 624
TAIL = N - NS * RB
TAIL0 = NS * RB


@functools.cache
def _mesh():
    return plsc.VectorSubcoreMesh(core_axis_name="c", subcore_axis_name="s",
                                  num_cores=NC, num_subcores=NS)


def _row_init(src_hbm, acc_sh, sid):
    base = sid * RB
    pltpu.sync_copy(src_hbm.at[pl.ds(base, RB)], acc_sh.at[pl.ds(base, RB)])

    @pl.when(sid == NS - 1)
    def _():
        pltpu.sync_copy(src_hbm.at[pl.ds(TAIL0, TAIL)],
                        acc_sh.at[pl.ds(TAIL0, TAIL)])


def _row_out(acc_sh, out_hbm, cid, sid):
    base = sid * RB
    obase = cid * N + sid * RB
    pltpu.sync_copy(acc_sh.at[pl.ds(base, RB)], out_hbm.at[pl.ds(obase, RB)])

    @pl.when(sid == NS - 1)
    def _():
        pltpu.sync_copy(acc_sh.at[pl.ds(TAIL0, TAIL)],
                        out_hbm.at[pl.ds(cid * N + TAIL0, TAIL)])


def _sc_agg_body(hp_hbm, src_hbm, dst_hbm, zeros_hbm, out_hbm,
                 src_a, dst_a, src_b, dst_b, rows_a, rows_b, acc_sh,
                 ia, ib, ga, gb):
    cid = lax.axis_index("c")
    sid = lax.axis_index("s")
    wid = sid * NC + cid
    ebase = wid * EWP

    @pl.when(cid == 0)
    def _():
        _row_init(hp_hbm, acc_sh, sid)

    @pl.when(cid == 1)
    def _():
        _row_init(zeros_hbm, acc_sh, sid)

    def idx_load(k, sv, dv, sem):
        pltpu.async_copy(src_hbm.at[pl.ds(ebase + k * C, C)], sv, sem)
        pltpu.async_copy(dst_hbm.at[pl.ds(ebase + k * C, C)], dv, sem)

    def idx_wait(sv, dv, sem):
        pltpu.make_async_copy(src_hbm.at[pl.ds(ebase, C)], sv, sem).wait()
        pltpu.make_async_copy(dst_hbm.at[pl.ds(ebase, C)], dv, sem).wait()

    plsc.subcore_barrier()

    idx_load(0, src_a, dst_a, ia)
    idx_wait(src_a, dst_a, ia)
    pltpu.async_copy(hp_hbm.at[src_a], rows_a, ga)
    idx_load(1, src_b, dst_b, ib)

    def body2(j, carry):
        k0 = 2 * j
        pltpu.make_async_copy(hp_hbm.at[src_a], rows_a, ga).wait()
        idx_wait(src_b, dst_b, ib)
        pltpu.async_copy(hp_hbm.at[src_b], rows_b, gb)
        pltpu.sync_copy(rows_a, acc_sh.at[dst_a], add=True)
        idx_load(k0 + 2, src_a, dst_a, ia)
        pltpu.make_async_copy(hp_hbm.at[src_b], rows_b, gb).wait()
        idx_wait(src_a, dst_a, ia)
        pltpu.async_copy(hp_hbm.at[src_a], rows_a, ga)
        pltpu.sync_copy(rows_b, acc_sh.at[dst_b], add=True)
        kn = jnp.minimum(k0 + 3, STEPS - 1)
        idx_load(kn, src_b, dst_b, ib)
        return carry

    lax.fori_loop(0, HALF, body2, 0)
    pltpu.make_async_copy(hp_hbm.at[src_a], rows_a, ga).wait()
    pltpu.sync_copy(rows_a, acc_sh.at[dst_a], add=True)
    idx_wait(src_b, dst_b, ib)
    plsc.subcore_barrier()
    _row_out(acc_sh, out_hbm, cid, sid)


def _sc_deg_body(dst_hbm, ones_hbm, zeros_hbm, out_hbm,
                 dst_a, dst_b, ones_v, acc_sh, ia, ib, sa, sb):
    cid = lax.axis_index("c")
    sid = lax.axis_index("s")
    wid = sid * NC + cid
    ebase = wid * EWP

    @pl.when(cid == 0)
    def _():
        _row_init(ones_hbm, acc_sh, sid)

    @pl.when(cid == 1)
    def _():
        _row_init(zeros_hbm, acc_sh, sid)

    pltpu.sync_copy(ones_hbm.at[pl.ds(0, C)], ones_v)

    def idx_load(k, dv, sem):
        pltpu.async_copy(dst_hbm.at[pl.ds(ebase + k * C, C)], dv, sem)

    def idx_wait(dv, sem):
        pltpu.make_async_copy(dst_hbm.at[pl.ds(ebase, C)], dv, sem).wait()

    plsc.subcore_barrier()

    idx_load(0, dst_a, ia)
    idx_wait(dst_a, ia)
    pltpu.async_copy(ones_v, acc_sh.at[dst_a], sa, add=True)
    idx_load(1, dst_b, ib)

    def body2(j, carry):
        k0 = 2 * j
        idx_wait(dst_b, ib)
        pltpu.async_copy(ones_v, acc_sh.at[dst_b], sb, add=True)
        pltpu.make_async_copy(ones_v, acc_sh.at[dst_a], sa).wait()
        idx_load(k0 + 2, dst_a, ia)
        idx_wait(dst_a, ia)
        pltpu.async_copy(ones_v, acc_sh.at[dst_a], sa, add=True)
        pltpu.make_async_copy(ones_v, acc_sh.at[dst_b], sb).wait()
        kn = jnp.minimum(k0 + 3, STEPS - 1)
        idx_load(kn, dst_b, ib)
        return carry

    lax.fori_loop(0, HALF, body2, 0)
    pltpu.make_async_copy(ones_v, acc_sh.at[dst_a], sa).wait()
    idx_wait(dst_b, ib)
    plsc.subcore_barrier()
    _row_out(acc_sh, out_hbm, cid, sid)


@functools.cache
def _sc_agg_call():
    return pl.kernel(
        _sc_agg_body,
        out_type=jax.ShapeDtypeStruct((2 * N, H), jnp.float32),
        mesh=_mesh(),
        scratch_types=[
            pltpu.VMEM((C,), jnp.int32),
            pltpu.VMEM((C,), jnp.int32),
            pltpu.VMEM((C,), jnp.int32),
            pltpu.VMEM((C,), jnp.int32),
            pltpu.VMEM((C, H), jnp.float32),
            pltpu.VMEM((C, H), jnp.float32),
            pltpu.VMEM_SHARED((NP, H), jnp.float32),
            pltpu.SemaphoreType.DMA,
            pltpu.SemaphoreType.DMA,
            pltpu.SemaphoreType.DMA,
            pltpu.SemaphoreType.DMA,
        ],
    )


def _sc_agg(hp, src, dst, zeros_h):
    return _sc_agg_call()(hp, src, dst, zeros_h)


@functools.cache
def _sc_deg_call():
    return pl.kernel(
        _sc_deg_body,
        out_type=jax.ShapeDtypeStruct((2 * N, H), jnp.float32),
        mesh=_mesh(),
        scratch_types=[
            pltpu.VMEM((C,), jnp.int32),
            pltpu.VMEM((C,), jnp.int32),
            pltpu.VMEM((C, H), jnp.float32),
            pltpu.VMEM_SHARED((NP, H), jnp.float32),
            pltpu.SemaphoreType.DMA,
            pltpu.SemaphoreType.DMA,
            pltpu.SemaphoreType.DMA,
            pltpu.SemaphoreType.DMA,
        ],
    )


def _sc_deg(dst, ones_h, zeros_h):
    return _sc_deg_call()(dst, ones_h, zeros_h)


def _tc_pre0_body(x_ref, w_ref, degp_ref, hp_ref, dinv_ref):
    deg = degp_ref[0, :, 0] + degp_ref[1, :, 0]
    dinv = lax.rsqrt(deg)[:, None]
    h = jnp.dot(x_ref[...], w_ref[...], preferred_element_type=jnp.float32)
    hp_ref[...] = h * dinv
    dinv_ref[...] = dinv


def _tc_pre0(x, w0, degp):
    return pl.pallas_call(
        _tc_pre0_body,
        out_shape=(jax.ShapeDtypeStruct((N, H), jnp.float32),
                   jax.ShapeDtypeStruct((N, 1), jnp.float32)),
    )(x, w0, degp)


def _ln_relu(pre, g, be):
    mu = jnp.mean(pre, axis=1, keepdims=True)
    d = pre - mu
    var = jnp.mean(d * d, axis=1, keepdims=True)
    y = d * lax.rsqrt(var + 1e-5) * g + be
    return jnp.maximum(y, 0.0)


def _tc_mid_body(accp_ref, dinv_ref, b_ref, g_ref, be_ref, w_ref, hp_ref):
    s = accp_ref[0] + accp_ref[1]
    pre = s * dinv_ref[...] + b_ref[...]
    y = _ln_relu(pre, g_ref[...], be_ref[...])
    hp_ref[...] = jnp.dot(y, w_ref[...],
                          preferred_element_type=jnp.float32) * dinv_ref[...]


def _tc_mid(accp, dinv, b, g, be, w_next):
    return pl.pallas_call(
        _tc_mid_body,
        out_shape=jax.ShapeDtypeStruct((N, H), jnp.float32),
    )(accp, dinv, b, g, be, w_next)


def _tc_fin_body(accp_ref, dinv_ref, b_ref, g_ref, be_ref, out_ref):
    s = accp_ref[0] + accp_ref[1]
    pre = s * dinv_ref[...] + b_ref[...]
    out_ref[...] = _ln_relu(pre, g_ref[...], be_ref[...])


def _tc_fin(accp, dinv, b, g, be):
    return pl.pallas_call(
        _tc_fin_body,
        out_shape=jax.ShapeDtypeStruct((N, H), jnp.float32),
    )(accp, dinv, b, g, be)


def kernel(node_features, W0, b0, W1, b1, W2, b2,
           g0, be0, g1, be1, g2, be2, edge_index):
    x = node_features.reshape(N, FIN)
    pad_src = jnp.arange(PAD, dtype=jnp.int32) % 16
    pad_dst = N + pad_src
    src = jnp.concatenate([edge_index[0], pad_src])
    dst = jnp.concatenate([edge_index[1], pad_dst])
    zeros_h = jnp.zeros((N, H), jnp.float32)
    ones_h = jnp.ones((N, H), jnp.float32)

    degp = _sc_deg(dst, ones_h, zeros_h).reshape(2, N, H)
    hp, dinv = _tc_pre0(x, W0, degp)

    b0r, b1r, b2r = (v.reshape(1, H) for v in (b0, b1, b2))
    g0r, g1r, g2r = (v.reshape(1, H) for v in (g0, g1, g2))
    be0r, be1r, be2r = (v.reshape(1, H) for v in (be0, be1, be2))

    accp = _sc_agg(hp, src, dst, zeros_h).reshape(2, N, H)
    hp = _tc_mid(accp, dinv, b0r, g0r, be0r, W1)
    accp = _sc_agg(hp, src, dst, zeros_h).reshape(2, N, H)
    hp = _tc_mid(accp, dinv, b1r, g1r, be1r, W2)
    accp = _sc_agg(hp, src, dst, zeros_h).reshape(2, N, H)
    out = _tc_fin(accp, dinv, b2r, g2r, be2r)
    return out.reshape(1, N, H)


# speedup vs baseline: 1.1239x; 1.0045x over previous
import functools

import jax
import jax.numpy as jnp
import numpy as np
from jax import lax
from jax.experimental import pallas as pl
from jax.experimental.pallas import tpu as pltpu
from jax.experimental.pallas import tpu_sc as plsc

N = 10000
E = 320000
H = 128
FIN = 8

NC, NS = 2, 16
NW = NC * NS
C = 96
STEPS = 105
EWP = C * STEPS
EP = EWP * NW
PAD = EP - E
NP = N + 16
HC = C // 2
HALF = (STEPS - 1) // 2
RB = 624
TAIL = N - NS * RB
TAIL0 = NS * RB


@functools.cache
def _mesh():
    return plsc.VectorSubcoreMesh(core_axis_name="c", subcore_axis_name="s",
                                  num_cores=NC, num_subcores=NS)


def _row_init(src_hbm, acc_sh, sid):
    base = sid * RB
    pltpu.sync_copy(src_hbm.at[pl.ds(base, RB)], acc_sh.at[pl.ds(base, RB)])

    @pl.when(sid == NS - 1)
    def _():
        pltpu.sync_copy(src_hbm.at[pl.ds(TAIL0, TAIL)],
                        acc_sh.at[pl.ds(TAIL0, TAIL)])


def _row_out(acc_sh, out_hbm, cid, sid):
    base = sid * RB
    obase = cid * N + sid * RB
    pltpu.sync_copy(acc_sh.at[pl.ds(base, RB)], out_hbm.at[pl.ds(obase, RB)])

    @pl.when(sid == NS - 1)
    def _():
        pltpu.sync_copy(acc_sh.at[pl.ds(TAIL0, TAIL)],
                        out_hbm.at[pl.ds(cid * N + TAIL0, TAIL)])


def _sc_agg_body(hp_hbm, src_hbm, dst_hbm, zeros_hbm, out_hbm,
                 src_a, dst_a, src_b, dst_b, rows_a, rows_b, acc_sh,
                 ia, ib, ga, gb):
    cid = lax.axis_index("c")
    sid = lax.axis_index("s")
    wid = sid * NC + cid
    ebase = wid * EWP

    @pl.when(cid == 0)
    def _():
        _row_init(hp_hbm, acc_sh, sid)

    @pl.when(cid == 1)
    def _():
        _row_init(zeros_hbm, acc_sh, sid)

    def idx_load(k, sv, dv, sem):
        pltpu.async_copy(src_hbm.at[pl.ds(ebase + k * C, C)], sv, sem)
        pltpu.async_copy(dst_hbm.at[pl.ds(ebase + k * C, C)], dv, sem)

    def idx_wait(sv, dv, sem):
        pltpu.make_async_copy(src_hbm.at[pl.ds(ebase, C)], sv, sem).wait()
        pltpu.make_async_copy(dst_hbm.at[pl.ds(ebase, C)], dv, sem).wait()

    plsc.subcore_barrier()

    idx_load(0, src_a, dst_a, ia)
    idx_wait(src_a, dst_a, ia)
    pltpu.async_copy(hp_hbm.at[src_a], rows_a, ga)
    idx_load(1, src_b, dst_b, ib)

    def body2(j, carry):
        k0 = 2 * j
        pltpu.make_async_copy(hp_hbm.at[src_a], rows_a, ga).wait()
        idx_wait(src_b, dst_b, ib)
        pltpu.async_copy(hp_hbm.at[src_b], rows_b, gb)
        pltpu.sync_copy(rows_a, acc_sh.at[dst_a], add=True)
        idx_load(k0 + 2, src_a, dst_a, ia)
        pltpu.make_async_copy(hp_hbm.at[src_b], rows_b, gb).wait()
        idx_wait(src_a, dst_a, ia)
        pltpu.async_copy(hp_hbm.at[src_a], rows_a, ga)
        pltpu.sync_copy(rows_b, acc_sh.at[dst_b], add=True)
        kn = jnp.minimum(k0 + 3, STEPS - 1)
        idx_load(kn, src_b, dst_b, ib)
        return carry

    lax.fori_loop(0, HALF, body2, 0)
    pltpu.make_async_copy(hp_hbm.at[src_a], rows_a, ga).wait()
    pltpu.sync_copy(rows_a, acc_sh.at[dst_a], add=True)
    idx_wait(src_b, dst_b, ib)
    plsc.subcore_barrier()
    _row_out(acc_sh, out_hbm, cid, sid)


def _sc_deg_body(dst_hbm, ones_hbm, zeros_hbm, out_hbm,
                 dst_a, dst_b, ones_v, acc_sh, ia, ib, sa, sb):
    cid = lax.axis_index("c")
    sid = lax.axis_index("s")
    wid = sid * NC + cid
    ebase = wid * EWP

    _row_init(zeros_hbm, acc_sh, sid)
    pltpu.sync_copy(ones_hbm, ones_v)

    def idx_load(k, dv, sem):
        pltpu.async_copy(dst_hbm.at[pl.ds(ebase + k * C, C)], dv, sem)

    def idx_wait(dv, sem):
        pltpu.make_async_copy(dst_hbm.at[pl.ds(ebase, C)], dv, sem).wait()

    plsc.subcore_barrier()

    idx_load(0, dst_a, ia)
    idx_wait(dst_a, ia)
    pltpu.async_copy(ones_v, acc_sh.at[dst_a], sa, add=True)
    idx_load(1, dst_b, ib)

    def body2(j, carry):
        k0 = 2 * j
        idx_wait(dst_b, ib)
        pltpu.async_copy(ones_v, acc_sh.at[dst_b], sb, add=True)
        pltpu.make_async_copy(ones_v, acc_sh.at[dst_a], sa).wait()
        idx_load(k0 + 2, dst_a, ia)
        idx_wait(dst_a, ia)
        pltpu.async_copy(ones_v, acc_sh.at[dst_a], sa, add=True)
        pltpu.make_async_copy(ones_v, acc_sh.at[dst_b], sb).wait()
        kn = jnp.minimum(k0 + 3, STEPS - 1)
        idx_load(kn, dst_b, ib)
        return carry

    lax.fori_loop(0, HALF, body2, 0)
    pltpu.make_async_copy(ones_v, acc_sh.at[dst_a], sa).wait()
    idx_wait(dst_b, ib)
    plsc.subcore_barrier()
    _row_out(acc_sh, out_hbm, cid, sid)


@functools.cache
def _sc_agg_call():
    return pl.kernel(
        _sc_agg_body,
        out_type=jax.ShapeDtypeStruct((2 * N, H), jnp.float32),
        mesh=_mesh(),
        scratch_types=[
            pltpu.VMEM((C,), jnp.int32),
            pltpu.VMEM((C,), jnp.int32),
            pltpu.VMEM((C,), jnp.int32),
            pltpu.VMEM((C,), jnp.int32),
            pltpu.VMEM((C, H), jnp.float32),
            pltpu.VMEM((C, H), jnp.float32),
            pltpu.VMEM_SHARED((NP, H), jnp.float32),
            pltpu.SemaphoreType.DMA,
            pltpu.SemaphoreType.DMA,
            pltpu.SemaphoreType.DMA,
            pltpu.SemaphoreType.DMA,
        ],
    )


def _sc_agg(hp, src, dst, zeros_h):
    return _sc_agg_call()(hp, src, dst, zeros_h)


@functools.cache
def _sc_deg_call():
    return pl.kernel(
        _sc_deg_body,
        out_type=jax.ShapeDtypeStruct((2 * N, H), jnp.float32),
        mesh=_mesh(),
        scratch_types=[
            pltpu.VMEM((C,), jnp.int32),
            pltpu.VMEM((C,), jnp.int32),
            pltpu.VMEM((C, H), jnp.float32),
            pltpu.VMEM_SHARED((NP, H), jnp.float32),
            pltpu.SemaphoreType.DMA,
            pltpu.SemaphoreType.DMA,
            pltpu.SemaphoreType.DMA,
            pltpu.SemaphoreType.DMA,
        ],
    )


def _sc_deg(dst, ones_h, zeros_h):
    return _sc_deg_call()(dst, ones_h, zeros_h)


def _tc_pre0_body(x_ref, w_ref, degp_ref, hp_ref, dinv_ref):
    deg = degp_ref[0, :, 0] + degp_ref[1, :, 0] + 1.0
    dinv = lax.rsqrt(deg)[:, None]
    h = jnp.dot(x_ref[...], w_ref[...], preferred_element_type=jnp.float32)
    hp_ref[...] = h * dinv
    dinv_ref[...] = dinv


def _tc_pre0(x, w0, degp):
    return pl.pallas_call(
        _tc_pre0_body,
        out_shape=(jax.ShapeDtypeStruct((N, H), jnp.float32),
                   jax.ShapeDtypeStruct((N, 1), jnp.float32)),
    )(x, w0, degp)


def _ln_relu(pre, g, be):
    mu = jnp.mean(pre, axis=1, keepdims=True)
    d = pre - mu
    var = jnp.mean(d * d, axis=1, keepdims=True)
    y = d * lax.rsqrt(var + 1e-5) * g + be
    return jnp.maximum(y, 0.0)


def _tc_mid_body(accp_ref, dinv_ref, b_ref, g_ref, be_ref, w_ref, hp_ref):
    s = accp_ref[0] + accp_ref[1]
    pre = s * dinv_ref[...] + b_ref[...]
    y = _ln_relu(pre, g_ref[...], be_ref[...])
    hp_ref[...] = jnp.dot(y, w_ref[...],
                          preferred_element_type=jnp.float32) * dinv_ref[...]


def _tc_mid(accp, dinv, b, g, be, w_next):
    return pl.pallas_call(
        _tc_mid_body,
        out_shape=jax.ShapeDtypeStruct((N, H), jnp.float32),
    )(accp, dinv, b, g, be, w_next)


def _tc_fin_body(accp_ref, dinv_ref, b_ref, g_ref, be_ref, out_ref):
    s = accp_ref[0] + accp_ref[1]
    pre = s * dinv_ref[...] + b_ref[...]
    out_ref[...] = _ln_relu(pre, g_ref[...], be_ref[...])


def _tc_fin(accp, dinv, b, g, be):
    return pl.pallas_call(
        _tc_fin_body,
        out_shape=jax.ShapeDtypeStruct((N, H), jnp.float32),
    )(accp, dinv, b, g, be)


def kernel(node_features, W0, b0, W1, b1, W2, b2,
           g0, be0, g1, be1, g2, be2, edge_index):
    x = node_features.reshape(N, FIN)
    pad_src = jnp.asarray(np.arange(PAD, dtype=np.int32) % 16)
    pad_dst = jnp.asarray(N + np.arange(PAD, dtype=np.int32) % 16)
    src = jnp.concatenate([edge_index[0], pad_src])
    dst = jnp.concatenate([edge_index[1], pad_dst])
    zeros_h = jnp.asarray(np.zeros((N, H), np.float32))
    ones_c = jnp.asarray(np.ones((C, H), np.float32))

    degp = _sc_deg(dst, ones_c, zeros_h).reshape(2, N, H)
    hp, dinv = _tc_pre0(x, W0, degp)

    b0r, b1r, b2r = (v.reshape(1, H) for v in (b0, b1, b2))
    g0r, g1r, g2r = (v.reshape(1, H) for v in (g0, g1, g2))
    be0r, be1r, be2r = (v.reshape(1, H) for v in (be0, be1, be2))

    accp = _sc_agg(hp, src, dst, zeros_h).reshape(2, N, H)
    hp = _tc_mid(accp, dinv, b0r, g0r, be0r, W1)
    accp = _sc_agg(hp, src, dst, zeros_h).reshape(2, N, H)
    hp = _tc_mid(accp, dinv, b1r, g1r, be1r, W2)
    accp = _sc_agg(hp, src, dst, zeros_h).reshape(2, N, H)
    out = _tc_fin(accp, dinv, b2r, g2r, be2r)
    return out.reshape(1, N, H)
